# baseline (speedup 1.0000x reference)
import functools

import jax
import jax.numpy as jnp
from jax import lax
from jax.experimental import pallas as pl
from jax.experimental.pallas import tpu as pltpu
from jax.experimental.pallas import tpu_sc as plsc

NC = 2
NS = 16
NW = NC * NS
IDX_W = 128


def _gather_call(n_rows, dim, k_streams):
    chunk = k_streams * IDX_W
    per_worker = n_rows // NW
    n_chunks = per_worker // chunk
    assert per_worker % chunk == 0

    mesh = plsc.VectorSubcoreMesh(
        core_axis_name="c", subcore_axis_name="s",
        num_cores=NC, num_subcores=NS,
    )

    @functools.partial(
        pl.kernel,
        out_type=jax.ShapeDtypeStruct((n_rows, dim), jnp.float32),
        mesh=mesh,
        compiler_params=pltpu.CompilerParams(use_tc_tiling_on_sc=False),
        scratch_types=[
            pltpu.VMEM((k_streams, IDX_W), jnp.int32),
            pltpu.VMEM((chunk, dim), jnp.float32),
            pltpu.SemaphoreType.DMA,
            pltpu.SemaphoreType.DMA,
        ],
    )
    def gather_kernel(ids_hbm, table_hbm, out_hbm, idx_v, rows_v, gsem, osem):
        wid = lax.axis_index("s") * NC + lax.axis_index("c")
        base = wid * n_chunks

        def body(c, carry):
            row0 = (base + c) * chunk
            irow0 = (base + c) * k_streams
            pltpu.sync_copy(ids_hbm.at[pl.ds(irow0, k_streams)], idx_v)
            copies = [
                pltpu.async_copy(
                    table_hbm.at[idx_v.at[j]],
                    rows_v.at[pl.ds(j * IDX_W, IDX_W)],
                    gsem,
                )
                for j in range(k_streams)
            ]
            for cp in copies:
                cp.wait()
            out_cp = pltpu.async_copy(
                rows_v, out_hbm.at[pl.ds(row0, chunk)], osem)
            out_cp.wait()
            return carry

        lax.fori_loop(0, n_chunks, body, 0)

    return gather_kernel


def kernel(token_ids, embeddings):
    batch, seq = token_ids.shape
    vocab, dim = embeddings.shape
    n_rows = batch * seq

    ids2d = token_ids.reshape(n_rows // IDX_W, IDX_W)
    flat = _gather_call(n_rows, dim, 4)(ids2d, embeddings)
    x = flat.reshape(batch, seq, dim)

    mask = jnp.ones((batch, seq), dtype=jnp.float32)
    time_steps = jnp.array(seq, dtype=jnp.int32)
    return (x, mask, time_steps)


# speedup vs baseline: 1.0476x; 1.0476x over previous
import functools

import jax
import jax.numpy as jnp
from jax import lax
from jax.experimental import pallas as pl
from jax.experimental.pallas import tpu as pltpu
from jax.experimental.pallas import tpu_sc as plsc

NC = 2
NS = 16
NW = NC * NS
IDX_W = 128


def _gather_call(n_rows, dim, k_streams, nbuf):
    chunk = k_streams * IDX_W
    per_worker = n_rows // NW
    n_chunks = per_worker // chunk
    n_irows = per_worker // IDX_W
    assert per_worker % chunk == 0 and n_chunks % nbuf == 0

    mesh = plsc.VectorSubcoreMesh(
        core_axis_name="c", subcore_axis_name="s",
        num_cores=NC, num_subcores=NS,
    )

    @functools.partial(
        pl.kernel,
        out_type=jax.ShapeDtypeStruct((n_rows, dim), jnp.float32),
        mesh=mesh,
        compiler_params=pltpu.CompilerParams(use_tc_tiling_on_sc=False),
        scratch_types=[
            pltpu.VMEM((n_irows, IDX_W), jnp.int32),
            [pltpu.VMEM((chunk, dim), jnp.float32) for _ in range(nbuf)],
            [pltpu.SemaphoreType.DMA for _ in range(nbuf)],
            [pltpu.SemaphoreType.DMA for _ in range(nbuf)],
        ],
    )
    def gather_kernel(ids_hbm, table_hbm, out_hbm, idx_v, rows_v, gsems, osems):
        wid = lax.axis_index("s") * NC + lax.axis_index("c")
        row_base = wid * per_worker
        irow_base = wid * n_irows

        pltpu.sync_copy(ids_hbm.at[pl.ds(irow_base, n_irows)], idx_v)

        def gather_copies(c, b):
            return [
                pltpu.make_async_copy(
                    table_hbm.at[idx_v.at[c * k_streams + j]],
                    rows_v[b].at[pl.ds(j * IDX_W, IDX_W)],
                    gsems[b],
                )
                for j in range(k_streams)
            ]

        for b in range(nbuf):
            for cp in gather_copies(b, b):
                cp.start()

        def slot(c, b):
            for cp in gather_copies(c, b):
                cp.wait()
            wb = pltpu.make_async_copy(
                rows_v[b], out_hbm.at[pl.ds(row_base + c * chunk, chunk)],
                osems[b])
            wb.start()
            nxt = c + nbuf

            @pl.when(nxt < n_chunks)
            def _():
                wb.wait()
                for cp in gather_copies(nxt, b):
                    cp.start()

        def body(g, carry):
            for b in range(nbuf):
                slot(g + b, b)
            return carry

        lax.fori_loop(0, n_chunks // nbuf, lambda g, cr: body(g * nbuf, cr),
                      0, unroll=False)
        for b in range(nbuf):
            pltpu.make_async_copy(
                rows_v[b], out_hbm.at[pl.ds(row_base, chunk)], osems[b]
            ).wait()

    return gather_kernel


def kernel(token_ids, embeddings):
    batch, seq = token_ids.shape
    vocab, dim = embeddings.shape
    n_rows = batch * seq

    ids2d = token_ids.reshape(n_rows // IDX_W, IDX_W)
    flat = _gather_call(n_rows, dim, 5, 2)(ids2d, embeddings)
    x = flat.reshape(batch, seq, dim)

    mask = jnp.ones((batch, seq), dtype=jnp.float32)
    time_steps = jnp.array(seq, dtype=jnp.int32)
    return (x, mask, time_steps)
